# baseline (speedup 1.0000x reference)
import functools

import jax
import jax.numpy as jnp
from jax import lax
from jax.experimental import pallas as pl
from jax.experimental.pallas import tpu as pltpu
from jax.experimental.pallas import tpu_sc as plsc

N_MET = 10000
N_RXN = 10000
E_SUB = 320000
E_ALL = 640000
HID = 128
MSG = 128

NC = 2
NS = 16
L = 16

CHPC = 64
HCOL = 64
B1 = 128
NB1 = 158
EP1 = NS * NB1 * B1
RPAD1 = 10232
NRPAD = 10240
STRIPE = NRPAD // NS

NPAD = 10240
HALF = NPAD // 2
B3 = 800
NB3 = (E_ALL // (NC * NS)) // B3


_SC_PARAMS = pltpu.CompilerParams(needs_layout_passes=False,
                                  use_tc_tiling_on_sc=False)


def _iota16():
    return lax.iota(jnp.int32, L)


def _sc1_body(conc_h, edata_h, u_h, w_h, b1_h, out_h, cnt_h,
              conc_v, u_v, w_v, b1_v, ebuf0, ebuf1, rxn0, rxn1, hbuf0, hbuf1,
              dump, acc_cnt, shared, semi0, semi1, sem0, sem1):
    cid = lax.axis_index("c")
    sid = lax.axis_index("s")
    nbbase = sid * NB1
    chan0 = cid * CHPC

    pltpu.sync_copy(conc_h, conc_v)
    pltpu.sync_copy(u_h, u_v)
    pltpu.sync_copy(w_h, w_v)
    pltpu.sync_copy(b1_h, b1_v)

    zv = jnp.zeros((L,), jnp.float32)
    ones = jnp.ones((L,), jnp.float32)
    zoffs = (0, 16, 32, 48)

    def _zstripe(r, _):
        for co in zoffs:
            dump[r, pl.ds(co, L)] = zv
        return 0
    lax.fori_loop(0, STRIPE, _zstripe, 0)
    pltpu.sync_copy(dump, shared.at[pl.ds(sid * STRIPE, STRIPE)])

    def _zcnt(jj, _):
        acc_cnt[pl.ds(jj * L, L)] = zv
        return 0
    lax.fori_loop(0, NRPAD // L, _zcnt, 0)

    plsc.subcore_barrier()

    pltpu.async_copy(edata_h.at[nbbase], ebuf0, semi0)
    pltpu.async_copy(edata_h.at[nbbase + 1], ebuf1, semi1)

    def _pair(j, _):
        for p, (ebuf, semi, hbuf, rxn_v, sem) in enumerate(
                ((ebuf0, semi0, hbuf0, rxn0, sem0),
                 (ebuf1, semi1, hbuf1, rxn1, sem1))):
            i = 2 * j + p

            pltpu.make_async_copy(edata_h.at[0], ebuf, semi).wait()
            a_g = []
            s_g = []
            for g in range(B1 // L):
                midx = ebuf[pl.ds(g * L, L)]
                a_g.append(plsc.load_gather(conc_v, [midx]))
                s_g.append(plsc.bitcast(ebuf[pl.ds(2 * B1 + g * L, L)],
                                        jnp.float32))
                ridx = ebuf[pl.ds(B1 + g * L, L)]
                rxn_v[pl.ds(g * L, L)] = ridx
                plsc.addupdate_scatter(acc_cnt, [ridx], ones)
            pltpu.async_copy(edata_h.at[nbbase + i + 2], ebuf, semi)

            @pl.when(j >= 1)
            def _wait():
                pltpu.make_async_copy(hbuf, shared.at[rxn_v], sem).wait()

            def _chan(c, _):
                cb = jnp.full((L,), c, jnp.int32)
                uc = u_v[chan0 + c]
                wc = w_v[chan0 + c]
                bc = b1_v[chan0 + c]
                for g in range(B1 // L):
                    t2 = a_g[g] * uc + (s_g[g] * wc + bc)
                    e = jnp.exp(t2)
                    th = 1.0 - 2.0 / (e + 1.0)
                    plsc.store_scatter(hbuf, [g * L + _iota16(), cb], th)
                return 0
            lax.fori_loop(0, CHPC, _chan, 0)
            pltpu.async_copy(hbuf, shared.at[rxn_v], sem, add=True)
        return 0
    lax.fori_loop(0, NB1 // 2, _pair, 0)
    pltpu.make_async_copy(edata_h.at[0], ebuf0, semi0).wait()
    pltpu.make_async_copy(edata_h.at[0], ebuf1, semi1).wait()
    pltpu.make_async_copy(hbuf0, shared.at[rxn0], sem0).wait()
    pltpu.make_async_copy(hbuf1, shared.at[rxn1], sem1).wait()

    @pl.when(cid == 0)
    def _dump_cnt():
        pltpu.sync_copy(acc_cnt, cnt_h.at[sid])

    plsc.subcore_barrier()
    pltpu.sync_copy(shared.at[pl.ds(sid * STRIPE, STRIPE)], dump)
    pltpu.sync_copy(dump, out_h.at[cid, pl.ds(sid * STRIPE, STRIPE)])


def _sc1(conc, edata, u2b, w2b, b2b):
    mesh = plsc.VectorSubcoreMesh(core_axis_name="c", subcore_axis_name="s",
                                  num_cores=NC, num_subcores=NS)
    f = pl.kernel(
        _sc1_body,
        out_type=(pltpu.HBM((NC, NRPAD, HCOL), jnp.float32),
                  pltpu.HBM((NS, NRPAD), jnp.float32)),
        mesh=mesh,
        compiler_params=_SC_PARAMS,
        scratch_types=[
            pltpu.VMEM((N_MET,), jnp.float32),
            pltpu.VMEM((HID, L), jnp.float32),
            pltpu.VMEM((HID, L), jnp.float32),
            pltpu.VMEM((HID, L), jnp.float32),
            pltpu.VMEM((3 * B1,), jnp.int32),
            pltpu.VMEM((3 * B1,), jnp.int32),
            pltpu.VMEM((B1,), jnp.int32),
            pltpu.VMEM((B1,), jnp.int32),
            pltpu.VMEM((B1, HCOL), jnp.float32),
            pltpu.VMEM((B1, HCOL), jnp.float32),
            pltpu.VMEM((STRIPE, HCOL), jnp.float32),
            pltpu.VMEM((NRPAD,), jnp.float32),
            pltpu.VMEM_SHARED((NRPAD, HCOL), jnp.float32),
            pltpu.SemaphoreType.DMA,
            pltpu.SemaphoreType.DMA,
            pltpu.SemaphoreType.DMA,
            pltpu.SemaphoreType.DMA,
        ],
    )
    return f(conc, edata, u2b, w2b, b2b)


BR2 = 2048


def _rate_body(hext_ref, cnt_ref, w2_ref, b2_ref, w3_ref, b3_ref, w4_ref, b4_ref, v_ref):
    h0 = hext_ref[0]
    h1 = hext_ref[1]
    cnt = jnp.sum(cnt_ref[...], axis=0)[:, None]
    w23 = jnp.dot(w2_ref[...], w3_ref[...], preferred_element_type=jnp.float32)
    b23 = jnp.dot(b2_ref[...], w3_ref[...], preferred_element_type=jnp.float32) + b3_ref[...]
    z = (jnp.dot(h0[:, :CHPC], w23[:CHPC, :], preferred_element_type=jnp.float32)
         + jnp.dot(h1[:, :CHPC], w23[CHPC:, :], preferred_element_type=jnp.float32)
         + cnt * b23)
    t = jnp.tanh(z)
    r = jnp.dot(t, w4_ref[...], preferred_element_type=jnp.float32) + b4_ref[...]
    v_ref[...] = jnp.maximum(r, 0.0) + jnp.log1p(jnp.exp(-jnp.abs(r)))


def _rates(Hext, cnt_parts, W2, b2, W3, b3, W4, b4):
    grid = (NRPAD // BR2,)
    return pl.pallas_call(
        _rate_body,
        grid=grid,
        in_specs=[
            pl.BlockSpec((NC, BR2, HCOL), lambda i: (0, i, 0)),
            pl.BlockSpec((NS, BR2), lambda i: (0, i)),
            pl.BlockSpec((MSG, HID), lambda i: (0, 0)),
            pl.BlockSpec((1, MSG), lambda i: (0, 0)),
            pl.BlockSpec((MSG, HID), lambda i: (0, 0)),
            pl.BlockSpec((1, HID), lambda i: (0, 0)),
            pl.BlockSpec((HID, 1), lambda i: (0, 0)),
            pl.BlockSpec((1, 1), lambda i: (0, 0)),
        ],
        out_specs=pl.BlockSpec((BR2, 1), lambda i: (i, 0)),
        out_shape=jax.ShapeDtypeStruct((NRPAD, 1), jnp.float32),
    )(Hext, cnt_parts, W2, b2, W3, b3, W4, b4)


def _sc3_body(v_h, edata_h, out_h, v_v, ebuf, acc):
    cid = lax.axis_index("c")
    sid = lax.axis_index("s")
    wid = cid * NS + sid
    nbbase = wid * NB3

    pltpu.sync_copy(v_h, v_v)
    zv = jnp.zeros((L,), jnp.float32)

    def _zero(jj, _):
        acc[pl.ds(jj * L, L)] = zv
        return 0
    lax.fori_loop(0, NPAD // L, _zero, 0)

    def _batch(i, _):
        pltpu.sync_copy(edata_h.at[nbbase + i], ebuf)

        def _grp(g, _):
            met = ebuf[pl.ds(g * L, L)]
            ridx = ebuf[pl.ds(B3 + g * L, L)]
            sto = plsc.bitcast(ebuf[pl.ds(2 * B3 + g * L, L)], jnp.float32)
            vv = plsc.load_gather(v_v, [ridx])
            plsc.addupdate_scatter(acc, [met], vv * sto)
            return 0
        lax.fori_loop(0, B3 // L, _grp, 0)
        return 0
    lax.fori_loop(0, NB3, _batch, 0)
    pltpu.sync_copy(acc, out_h.at[wid])


def _sc3(v, edata3):
    mesh = plsc.VectorSubcoreMesh(core_axis_name="c", subcore_axis_name="s",
                                  num_cores=NC, num_subcores=NS)
    f = pl.kernel(
        _sc3_body,
        out_type=pltpu.HBM((NC * NS, NPAD), jnp.float32),
        mesh=mesh,
        compiler_params=_SC_PARAMS,
        scratch_types=[
            pltpu.VMEM((N_RXN,), jnp.float32),
            pltpu.VMEM((3 * B3,), jnp.int32),
            pltpu.VMEM((NPAD,), jnp.float32),
        ],
    )
    return f(v, edata3)


def _red_body(p_ref, o_ref):
    o_ref[...] = jnp.sum(p_ref[...], axis=0, keepdims=True)


def _reduce_parts(part):
    return pl.pallas_call(
        _red_body,
        grid=(1,),
        in_specs=[pl.BlockSpec((NC * NS, NPAD), lambda i: (0, 0))],
        out_specs=pl.BlockSpec((1, NPAD), lambda i: (0, 0)),
        out_shape=jax.ShapeDtypeStruct((1, NPAD), jnp.float32),
    )(part)


def kernel(x, met_sub, rxn_sub, sto_sub, met_all, rxn_all, sto_all,
           W1, b1, W2, b2, W3, b3, W4, b4):
    conc = x[:, 3]
    met_sub = met_sub.astype(jnp.int32)
    rxn_sub = rxn_sub.astype(jnp.int32)
    met_all = met_all.astype(jnp.int32)
    rxn_all = rxn_all.astype(jnp.int32)
    u2b = jnp.broadcast_to((2.0 * W1[0])[:, None], (HID, L))
    w2b = jnp.broadcast_to((2.0 * W1[1])[:, None], (HID, L))
    b2b = jnp.broadcast_to((2.0 * b1)[:, None], (HID, L))
    npad1 = EP1 + 2 * B1 - E_SUB
    met_p = jnp.concatenate([met_sub, jnp.zeros((npad1,), jnp.int32)])
    rxn_p = jnp.concatenate([rxn_sub, jnp.full((npad1,), RPAD1, jnp.int32)])
    sto_p = jnp.concatenate([sto_sub, jnp.zeros((npad1,), jnp.float32)])
    sto_bits = lax.bitcast_convert_type(sto_p, jnp.int32)
    edata = jnp.concatenate([met_p.reshape(-1, B1), rxn_p.reshape(-1, B1),
                             sto_bits.reshape(-1, B1)], axis=1)
    Hext, cnt_parts = _sc1(conc, edata, u2b, w2b, b2b)
    v2d = _rates(Hext, cnt_parts, W2, b2[None, :], W3, b3[None, :], W4, b4[None, :])
    stoa_bits = lax.bitcast_convert_type(sto_all, jnp.int32)
    edata3 = jnp.concatenate([met_all.reshape(-1, B3), rxn_all.reshape(-1, B3),
                              stoa_bits.reshape(-1, B3)], axis=1)
    part = _sc3(v2d[:N_RXN, 0], edata3)
    tot = _reduce_parts(part)
    return tot[0, :N_MET][:, None]


# speedup vs baseline: 2.0587x; 2.0587x over previous
import functools

import jax
import jax.numpy as jnp
from jax import lax
from jax.experimental import pallas as pl
from jax.experimental.pallas import tpu as pltpu
from jax.experimental.pallas import tpu_sc as plsc

N_MET = 10000
N_RXN = 10000
E_SUB = 320000
E_ALL = 640000
HID = 128
MSG = 128

NC = 2
NS = 16
L = 16

CHPC = 64
HCOL = 72
B1 = 128
NB1 = 158
EP1 = NS * NB1 * B1
RPAD1 = 10232
NRPAD = 10240
STRIPE = NRPAD // NS

NPAD = 10240
HALF = NPAD // 2
B3 = 800
NB3 = (E_ALL // (NC * NS)) // B3


_SC_PARAMS = pltpu.CompilerParams(needs_layout_passes=False,
                                  use_tc_tiling_on_sc=False)


def _iota16():
    return lax.iota(jnp.int32, L)


def _sc1_body(conc_h, edata_h, u_h, w_h, b1_h, out_h,
              conc_v, u_v, w_v, b1_v, ebuf0, ebuf1, rxn0, rxn1, hbuf0, hbuf1,
              dump, shared, semi0, semi1, sem0, sem1):
    cid = lax.axis_index("c")
    sid = lax.axis_index("s")
    nbbase = sid * NB1
    chan0 = cid * CHPC

    pltpu.sync_copy(conc_h, conc_v)
    pltpu.sync_copy(u_h, u_v)
    pltpu.sync_copy(w_h, w_v)
    pltpu.sync_copy(b1_h, b1_v)

    zv = jnp.zeros((L,), jnp.float32)
    zoffs = (0, 16, 32, 48, HCOL - L)

    def _zstripe(r, _):
        for co in zoffs:
            dump[r, pl.ds(co, L)] = zv
        return 0
    lax.fori_loop(0, STRIPE, _zstripe, 0)
    pltpu.sync_copy(dump, shared.at[pl.ds(sid * STRIPE, STRIPE)])

    ones = jnp.ones((L,), jnp.float32)
    ccnt = jnp.full((L,), CHPC, jnp.int32)
    for hbuf in (hbuf0, hbuf1):
        def _zrow(r, _):
            for co in zoffs:
                hbuf[r, pl.ds(co, L)] = zv
            return 0
        lax.fori_loop(0, B1, _zrow, 0)
        for g in range(B1 // L):
            plsc.store_scatter(hbuf, [g * L + _iota16(), ccnt], ones)

    plsc.subcore_barrier()

    pltpu.async_copy(edata_h.at[nbbase], ebuf0, semi0)
    pltpu.async_copy(edata_h.at[nbbase + 1], ebuf1, semi1)

    def _pair(j, _):
        for p, (ebuf, semi, hbuf, rxn_v, sem) in enumerate(
                ((ebuf0, semi0, hbuf0, rxn0, sem0),
                 (ebuf1, semi1, hbuf1, rxn1, sem1))):
            i = 2 * j + p

            pltpu.make_async_copy(edata_h.at[0], ebuf, semi).wait()
            a_g = []
            s_g = []
            for g in range(B1 // L):
                midx = ebuf[pl.ds(g * L, L)]
                a_g.append(plsc.load_gather(conc_v, [midx]))
                s_g.append(plsc.bitcast(ebuf[pl.ds(2 * B1 + g * L, L)],
                                        jnp.float32))
                rxn_v[pl.ds(g * L, L)] = ebuf[pl.ds(B1 + g * L, L)]
            pltpu.async_copy(edata_h.at[nbbase + i + 2], ebuf, semi)

            @pl.when(j >= 1)
            def _wait():
                pltpu.make_async_copy(hbuf, shared.at[rxn_v], sem).wait()

            def _chan(c, _):
                cb = jnp.full((L,), c, jnp.int32)
                uc = u_v[chan0 + c]
                wc = w_v[chan0 + c]
                bc = b1_v[chan0 + c]
                for g in range(B1 // L):
                    t2 = a_g[g] * uc + (s_g[g] * wc + bc)
                    e = jnp.exp(t2)
                    th = 1.0 - 2.0 / (e + 1.0)
                    plsc.store_scatter(hbuf, [g * L + _iota16(), cb], th)
                return 0
            lax.fori_loop(0, CHPC, _chan, 0)
            pltpu.async_copy(hbuf, shared.at[rxn_v], sem, add=True)
        return 0
    lax.fori_loop(0, NB1 // 2, _pair, 0)
    pltpu.make_async_copy(edata_h.at[0], ebuf0, semi0).wait()
    pltpu.make_async_copy(edata_h.at[0], ebuf1, semi1).wait()
    pltpu.make_async_copy(hbuf0, shared.at[rxn0], sem0).wait()
    pltpu.make_async_copy(hbuf1, shared.at[rxn1], sem1).wait()

    plsc.subcore_barrier()
    pltpu.sync_copy(shared.at[pl.ds(sid * STRIPE, STRIPE)], dump)
    pltpu.sync_copy(dump, out_h.at[cid, pl.ds(sid * STRIPE, STRIPE)])


def _sc1(conc, edata, u2b, w2b, b2b):
    mesh = plsc.VectorSubcoreMesh(core_axis_name="c", subcore_axis_name="s",
                                  num_cores=NC, num_subcores=NS)
    f = pl.kernel(
        _sc1_body,
        out_type=pltpu.HBM((NC, NRPAD, HCOL), jnp.float32),
        mesh=mesh,
        compiler_params=_SC_PARAMS,
        scratch_types=[
            pltpu.VMEM((N_MET,), jnp.float32),
            pltpu.VMEM((HID, L), jnp.float32),
            pltpu.VMEM((HID, L), jnp.float32),
            pltpu.VMEM((HID, L), jnp.float32),
            pltpu.VMEM((3 * B1,), jnp.int32),
            pltpu.VMEM((3 * B1,), jnp.int32),
            pltpu.VMEM((B1,), jnp.int32),
            pltpu.VMEM((B1,), jnp.int32),
            pltpu.VMEM((B1, HCOL), jnp.float32),
            pltpu.VMEM((B1, HCOL), jnp.float32),
            pltpu.VMEM((STRIPE, HCOL), jnp.float32),
            pltpu.VMEM_SHARED((NRPAD, HCOL), jnp.float32),
            pltpu.SemaphoreType.DMA,
            pltpu.SemaphoreType.DMA,
            pltpu.SemaphoreType.DMA,
            pltpu.SemaphoreType.DMA,
        ],
    )
    return f(conc, edata, u2b, w2b, b2b)


BR2 = 2048


def _rate_body(hext_ref, w2_ref, b2_ref, w3_ref, b3_ref, w4_ref, b4_ref, v_ref):
    h0 = hext_ref[0]
    h1 = hext_ref[1]
    cnt = h0[:, CHPC:CHPC + 1]
    w23 = jnp.dot(w2_ref[...], w3_ref[...], preferred_element_type=jnp.float32)
    b23 = jnp.dot(b2_ref[...], w3_ref[...], preferred_element_type=jnp.float32) + b3_ref[...]
    z = (jnp.dot(h0[:, :CHPC], w23[:CHPC, :], preferred_element_type=jnp.float32)
         + jnp.dot(h1[:, :CHPC], w23[CHPC:, :], preferred_element_type=jnp.float32)
         + cnt * b23)
    t = jnp.tanh(z)
    r = jnp.dot(t, w4_ref[...], preferred_element_type=jnp.float32) + b4_ref[...]
    v_ref[...] = jnp.maximum(r, 0.0) + jnp.log1p(jnp.exp(-jnp.abs(r)))


def _rates(Hext, W2, b2, W3, b3, W4, b4):
    grid = (NRPAD // BR2,)
    return pl.pallas_call(
        _rate_body,
        grid=grid,
        in_specs=[
            pl.BlockSpec((NC, BR2, HCOL), lambda i: (0, i, 0)),
            pl.BlockSpec((MSG, HID), lambda i: (0, 0)),
            pl.BlockSpec((1, MSG), lambda i: (0, 0)),
            pl.BlockSpec((MSG, HID), lambda i: (0, 0)),
            pl.BlockSpec((1, HID), lambda i: (0, 0)),
            pl.BlockSpec((HID, 1), lambda i: (0, 0)),
            pl.BlockSpec((1, 1), lambda i: (0, 0)),
        ],
        out_specs=pl.BlockSpec((BR2, 1), lambda i: (i, 0)),
        out_shape=jax.ShapeDtypeStruct((NRPAD, 1), jnp.float32),
    )(Hext, W2, b2, W3, b3, W4, b4)


def _sc3_body(v_h, edata_h, out_h, v_v, ebuf, acc):
    cid = lax.axis_index("c")
    sid = lax.axis_index("s")
    wid = cid * NS + sid
    nbbase = wid * NB3

    pltpu.sync_copy(v_h, v_v)
    zv = jnp.zeros((L,), jnp.float32)

    def _zero(jj, _):
        acc[pl.ds(jj * L, L)] = zv
        return 0
    lax.fori_loop(0, NPAD // L, _zero, 0)

    def _batch(i, _):
        pltpu.sync_copy(edata_h.at[nbbase + i], ebuf)

        def _grp(g, _):
            met = ebuf[pl.ds(g * L, L)]
            ridx = ebuf[pl.ds(B3 + g * L, L)]
            sto = plsc.bitcast(ebuf[pl.ds(2 * B3 + g * L, L)], jnp.float32)
            vv = plsc.load_gather(v_v, [ridx])
            plsc.addupdate_scatter(acc, [met], vv * sto)
            return 0
        lax.fori_loop(0, B3 // L, _grp, 0)
        return 0
    lax.fori_loop(0, NB3, _batch, 0)
    pltpu.sync_copy(acc, out_h.at[wid])


def _sc3(v, edata3):
    mesh = plsc.VectorSubcoreMesh(core_axis_name="c", subcore_axis_name="s",
                                  num_cores=NC, num_subcores=NS)
    f = pl.kernel(
        _sc3_body,
        out_type=pltpu.HBM((NC * NS, NPAD), jnp.float32),
        mesh=mesh,
        compiler_params=_SC_PARAMS,
        scratch_types=[
            pltpu.VMEM((N_RXN,), jnp.float32),
            pltpu.VMEM((3 * B3,), jnp.int32),
            pltpu.VMEM((NPAD,), jnp.float32),
        ],
    )
    return f(v, edata3)


def _red_body(p_ref, o_ref):
    o_ref[...] = jnp.sum(p_ref[...], axis=0, keepdims=True)


def _reduce_parts(part):
    return pl.pallas_call(
        _red_body,
        grid=(1,),
        in_specs=[pl.BlockSpec((NC * NS, NPAD), lambda i: (0, 0))],
        out_specs=pl.BlockSpec((1, NPAD), lambda i: (0, 0)),
        out_shape=jax.ShapeDtypeStruct((1, NPAD), jnp.float32),
    )(part)


def kernel(x, met_sub, rxn_sub, sto_sub, met_all, rxn_all, sto_all,
           W1, b1, W2, b2, W3, b3, W4, b4):
    conc = x[:, 3]
    met_sub = met_sub.astype(jnp.int32)
    rxn_sub = rxn_sub.astype(jnp.int32)
    met_all = met_all.astype(jnp.int32)
    rxn_all = rxn_all.astype(jnp.int32)
    u2b = jnp.broadcast_to((2.0 * W1[0])[:, None], (HID, L))
    w2b = jnp.broadcast_to((2.0 * W1[1])[:, None], (HID, L))
    b2b = jnp.broadcast_to((2.0 * b1)[:, None], (HID, L))
    npad1 = EP1 + 2 * B1 - E_SUB
    met_p = jnp.concatenate([met_sub, jnp.zeros((npad1,), jnp.int32)])
    rxn_p = jnp.concatenate([rxn_sub, jnp.full((npad1,), RPAD1, jnp.int32)])
    sto_p = jnp.concatenate([sto_sub, jnp.zeros((npad1,), jnp.float32)])
    sto_bits = lax.bitcast_convert_type(sto_p, jnp.int32)
    edata = jnp.concatenate([met_p.reshape(-1, B1), rxn_p.reshape(-1, B1),
                             sto_bits.reshape(-1, B1)], axis=1)
    Hext = _sc1(conc, edata, u2b, w2b, b2b)
    v2d = _rates(Hext, W2, b2[None, :], W3, b3[None, :], W4, b4[None, :])
    stoa_bits = lax.bitcast_convert_type(sto_all, jnp.int32)
    edata3 = jnp.concatenate([met_all.reshape(-1, B3), rxn_all.reshape(-1, B3),
                              stoa_bits.reshape(-1, B3)], axis=1)
    part = _sc3(v2d[:N_RXN, 0], edata3)
    tot = _reduce_parts(part)
    return tot[0, :N_MET][:, None]


# speedup vs baseline: 2.0598x; 1.0005x over previous
import jax
import jax.numpy as jnp
from jax import lax
from jax.experimental import pallas as pl
from jax.experimental.pallas import tpu as pltpu
from jax.experimental.pallas import tpu_sc as plsc

N_MET = 10000
N_RXN = 10000
E_SUB = 320000
E_ALL = 640000
HID = 128
MSG = 128

NC = 2
NS = 16
L = 16

CHPC = 64
HCOL = 72
B1 = 128
NB1 = 158
EP1 = NS * NB1 * B1
RPAD1 = 10232
NRPAD = 10240
STRIPE = NRPAD // NS

NPAD = 10240
B3 = 800
NB3 = (E_ALL // (NC * NS)) // B3


_SC_PARAMS = pltpu.CompilerParams(needs_layout_passes=False,
                                  use_tc_tiling_on_sc=False)


def _iota16():
    return lax.iota(jnp.int32, L)


def _sc1_body(conc_h, edata_h, u_h, w_h, b1_h, out_h,
              conc_v, u_v, w_v, b1_v, ebuf0, ebuf1, rxn0, rxn1, hbuf0, hbuf1,
              dump, shared, semi0, semi1, sem0, sem1):
    cid = lax.axis_index("c")
    sid = lax.axis_index("s")
    nbbase = sid * NB1
    chan0 = cid * CHPC

    pltpu.sync_copy(conc_h, conc_v)
    pltpu.sync_copy(u_h, u_v)
    pltpu.sync_copy(w_h, w_v)
    pltpu.sync_copy(b1_h, b1_v)

    zv = jnp.zeros((L,), jnp.float32)
    zoffs = (0, 16, 32, 48, HCOL - L)

    def _zstripe(r, _):
        for co in zoffs:
            dump[r, pl.ds(co, L)] = zv
        return 0
    lax.fori_loop(0, STRIPE, _zstripe, 0)
    pltpu.sync_copy(dump, shared.at[pl.ds(sid * STRIPE, STRIPE)])

    ones = jnp.ones((L,), jnp.float32)
    ccnt = jnp.full((L,), CHPC, jnp.int32)
    for hbuf in (hbuf0, hbuf1):
        def _zrow(r, _):
            for co in zoffs:
                hbuf[r, pl.ds(co, L)] = zv
            return 0
        lax.fori_loop(0, B1, _zrow, 0)
        for g in range(B1 // L):
            plsc.store_scatter(hbuf, [g * L + _iota16(), ccnt], ones)

    plsc.subcore_barrier()

    pltpu.async_copy(edata_h.at[nbbase], ebuf0, semi0)
    pltpu.async_copy(edata_h.at[nbbase + 1], ebuf1, semi1)

    def _pair(j, _):
        for p, (ebuf, semi, hbuf, rxn_v, sem) in enumerate(
                ((ebuf0, semi0, hbuf0, rxn0, sem0),
                 (ebuf1, semi1, hbuf1, rxn1, sem1))):
            i = 2 * j + p

            pltpu.make_async_copy(edata_h.at[0], ebuf, semi).wait()
            a_g = []
            s_g = []
            for g in range(B1 // L):
                midx = ebuf[pl.ds(g * L, L)]
                a_g.append(plsc.load_gather(conc_v, [midx]))
                s_g.append(plsc.bitcast(ebuf[pl.ds(2 * B1 + g * L, L)],
                                        jnp.float32))
                rxn_v[pl.ds(g * L, L)] = ebuf[pl.ds(B1 + g * L, L)]
            pltpu.async_copy(edata_h.at[nbbase + i + 2], ebuf, semi)

            @pl.when(j >= 1)
            def _wait():
                pltpu.make_async_copy(hbuf, shared.at[rxn_v], sem).wait()

            def _chan(c, _):
                cb = jnp.full((L,), c, jnp.int32)
                uc = u_v[chan0 + c]
                wc = w_v[chan0 + c]
                bc = b1_v[chan0 + c]
                for g in range(B1 // L):
                    t2 = a_g[g] * uc + (s_g[g] * wc + bc)
                    e = jnp.exp(t2)
                    th = 1.0 - 2.0 / (e + 1.0)
                    plsc.store_scatter(hbuf, [g * L + _iota16(), cb], th)
                return 0
            lax.fori_loop(0, CHPC, _chan, 0)
            pltpu.async_copy(hbuf, shared.at[rxn_v], sem, add=True)
        return 0
    lax.fori_loop(0, NB1 // 2, _pair, 0)
    pltpu.make_async_copy(edata_h.at[0], ebuf0, semi0).wait()
    pltpu.make_async_copy(edata_h.at[0], ebuf1, semi1).wait()
    pltpu.make_async_copy(hbuf0, shared.at[rxn0], sem0).wait()
    pltpu.make_async_copy(hbuf1, shared.at[rxn1], sem1).wait()

    plsc.subcore_barrier()
    pltpu.sync_copy(shared.at[pl.ds(sid * STRIPE, STRIPE)], dump)
    pltpu.sync_copy(dump, out_h.at[cid, pl.ds(sid * STRIPE, STRIPE)])


def _sc1(conc, edata, u2b, w2b, b2b):
    mesh = plsc.VectorSubcoreMesh(core_axis_name="c", subcore_axis_name="s",
                                  num_cores=NC, num_subcores=NS)
    f = pl.kernel(
        _sc1_body,
        out_type=pltpu.HBM((NC, NRPAD, HCOL), jnp.float32),
        mesh=mesh,
        compiler_params=_SC_PARAMS,
        scratch_types=[
            pltpu.VMEM((N_MET,), jnp.float32),
            pltpu.VMEM((HID, L), jnp.float32),
            pltpu.VMEM((HID, L), jnp.float32),
            pltpu.VMEM((HID, L), jnp.float32),
            pltpu.VMEM((3 * B1,), jnp.int32),
            pltpu.VMEM((3 * B1,), jnp.int32),
            pltpu.VMEM((B1,), jnp.int32),
            pltpu.VMEM((B1,), jnp.int32),
            pltpu.VMEM((B1, HCOL), jnp.float32),
            pltpu.VMEM((B1, HCOL), jnp.float32),
            pltpu.VMEM((STRIPE, HCOL), jnp.float32),
            pltpu.VMEM_SHARED((NRPAD, HCOL), jnp.float32),
            pltpu.SemaphoreType.DMA,
            pltpu.SemaphoreType.DMA,
            pltpu.SemaphoreType.DMA,
            pltpu.SemaphoreType.DMA,
        ],
    )
    return f(conc, edata, u2b, w2b, b2b)


BR2 = 2048


def _rate_body(hext_ref, w2_ref, b2_ref, w3_ref, b3_ref, w4_ref, b4_ref, v_ref):
    h0 = hext_ref[0]
    h1 = hext_ref[1]
    cnt = h0[:, CHPC:CHPC + 1]
    w23 = jnp.dot(w2_ref[...], w3_ref[...], preferred_element_type=jnp.float32)
    b23 = jnp.dot(b2_ref[...], w3_ref[...], preferred_element_type=jnp.float32) + b3_ref[...]
    z = (jnp.dot(h0[:, :CHPC], w23[:CHPC, :], preferred_element_type=jnp.float32)
         + jnp.dot(h1[:, :CHPC], w23[CHPC:, :], preferred_element_type=jnp.float32)
         + cnt * b23)
    t = jnp.tanh(z)
    r = jnp.dot(t, w4_ref[...], preferred_element_type=jnp.float32) + b4_ref[...]
    v_ref[...] = jnp.maximum(r, 0.0) + jnp.log1p(jnp.exp(-jnp.abs(r)))


def _rates(Hext, W2, b2, W3, b3, W4, b4):
    grid = (NRPAD // BR2,)
    return pl.pallas_call(
        _rate_body,
        grid=grid,
        in_specs=[
            pl.BlockSpec((NC, BR2, HCOL), lambda i: (0, i, 0)),
            pl.BlockSpec((MSG, HID), lambda i: (0, 0)),
            pl.BlockSpec((1, MSG), lambda i: (0, 0)),
            pl.BlockSpec((MSG, HID), lambda i: (0, 0)),
            pl.BlockSpec((1, HID), lambda i: (0, 0)),
            pl.BlockSpec((HID, 1), lambda i: (0, 0)),
            pl.BlockSpec((1, 1), lambda i: (0, 0)),
        ],
        out_specs=pl.BlockSpec((BR2, 1), lambda i: (i, 0)),
        out_shape=jax.ShapeDtypeStruct((NRPAD, 1), jnp.float32),
    )(Hext, W2, b2, W3, b3, W4, b4)


def _sc3_body(v_h, edata_h, out_h, v_v, ebuf, acc):
    cid = lax.axis_index("c")
    sid = lax.axis_index("s")
    wid = cid * NS + sid
    nbbase = wid * NB3

    pltpu.sync_copy(v_h, v_v)
    zv = jnp.zeros((L,), jnp.float32)

    def _zero(jj, _):
        acc[pl.ds(jj * L, L)] = zv
        return 0
    lax.fori_loop(0, NPAD // L, _zero, 0)

    def _batch(i, _):
        pltpu.sync_copy(edata_h.at[nbbase + i], ebuf)

        def _grp(g, _):
            met = ebuf[pl.ds(g * L, L)]
            ridx = ebuf[pl.ds(B3 + g * L, L)]
            sto = plsc.bitcast(ebuf[pl.ds(2 * B3 + g * L, L)], jnp.float32)
            vv = plsc.load_gather(v_v, [ridx])
            plsc.addupdate_scatter(acc, [met], vv * sto)
            return 0
        lax.fori_loop(0, B3 // L, _grp, 0)
        return 0
    lax.fori_loop(0, NB3, _batch, 0)
    pltpu.sync_copy(acc, out_h.at[wid])


def _sc3(v, edata3):
    mesh = plsc.VectorSubcoreMesh(core_axis_name="c", subcore_axis_name="s",
                                  num_cores=NC, num_subcores=NS)
    f = pl.kernel(
        _sc3_body,
        out_type=pltpu.HBM((NC * NS, NPAD), jnp.float32),
        mesh=mesh,
        compiler_params=_SC_PARAMS,
        scratch_types=[
            pltpu.VMEM((N_RXN,), jnp.float32),
            pltpu.VMEM((3 * B3,), jnp.int32),
            pltpu.VMEM((NPAD,), jnp.float32),
        ],
    )
    return f(v, edata3)


def _red_body(p_ref, o_ref):
    o_ref[...] = jnp.sum(p_ref[...], axis=0, keepdims=True)


def _reduce_parts(part):
    return pl.pallas_call(
        _red_body,
        grid=(1,),
        in_specs=[pl.BlockSpec((NC * NS, NPAD), lambda i: (0, 0))],
        out_specs=pl.BlockSpec((1, NPAD), lambda i: (0, 0)),
        out_shape=jax.ShapeDtypeStruct((1, NPAD), jnp.float32),
    )(part)


def kernel(x, met_sub, rxn_sub, sto_sub, met_all, rxn_all, sto_all,
           W1, b1, W2, b2, W3, b3, W4, b4):
    conc = x[:, 3]
    met_sub = met_sub.astype(jnp.int32)
    rxn_sub = rxn_sub.astype(jnp.int32)
    met_all = met_all.astype(jnp.int32)
    rxn_all = rxn_all.astype(jnp.int32)
    u2b = jnp.broadcast_to((2.0 * W1[0])[:, None], (HID, L))
    w2b = jnp.broadcast_to((2.0 * W1[1])[:, None], (HID, L))
    b2b = jnp.broadcast_to((2.0 * b1)[:, None], (HID, L))
    npad1 = EP1 + 2 * B1 - E_SUB
    met_p = jnp.concatenate([met_sub, jnp.zeros((npad1,), jnp.int32)])
    rxn_p = jnp.concatenate([rxn_sub, jnp.full((npad1,), RPAD1, jnp.int32)])
    sto_p = jnp.concatenate([sto_sub, jnp.zeros((npad1,), jnp.float32)])
    sto_bits = lax.bitcast_convert_type(sto_p, jnp.int32)
    edata = jnp.concatenate([met_p.reshape(-1, B1), rxn_p.reshape(-1, B1),
                             sto_bits.reshape(-1, B1)], axis=1)
    Hext = _sc1(conc, edata, u2b, w2b, b2b)
    v2d = _rates(Hext, W2, b2[None, :], W3, b3[None, :], W4, b4[None, :])
    stoa_bits = lax.bitcast_convert_type(sto_all, jnp.int32)
    edata3 = jnp.concatenate([met_all.reshape(-1, B3), rxn_all.reshape(-1, B3),
                              stoa_bits.reshape(-1, B3)], axis=1)
    part = _sc3(v2d[:N_RXN, 0], edata3)
    tot = _reduce_parts(part)
    return tot[0, :N_MET][:, None]
